# baseline (device time: 446058 ns/iter reference)
import functools

import jax
import jax.numpy as jnp
from jax import lax
from jax.experimental import pallas as pl
from jax.experimental.pallas import tpu as pltpu

N_DEV = 4
USE_FLASH = True

f32 = jnp.float32
bf16 = jnp.bfloat16


def _ring_allreduce_fused(p, res, gate, collective_id, Wo=None):
    S, D = p.shape
    C = S // N_DEV
    Dh_ = D // 2

    def body(*refs):
        if Wo is None:
            (p_ref, res_ref, gate_ref, out_ref,
             rs_send_r, rs_recv_r, ag_r,
             rs_send_l, rs_recv_l, ag_l,
             ssem_r, rsem_r, ssem_l, rsem_l) = refs
        else:
            (a_ref, wo_ref, res_ref, gate_ref, out_ref, p_scr,
             rs_send_r, rs_recv_r, ag_r,
             rs_send_l, rs_recv_l, ag_l,
             ssem_r, rsem_r, ssem_l, rsem_l) = refs
            p_ref = p_scr
        my = lax.axis_index("i")
        left = lax.rem(my + N_DEV - 1, N_DEV)
        right = lax.rem(my + 1, N_DEV)

        def gemm_chunk(idx):
            if Wo is None:
                return
            p_scr[pl.ds(idx * C, C), :] = lax.dot_general(
                a_ref[pl.ds(idx * C, C), :], wo_ref[...],
                (((1,), (0,)), ((), ())),
                preferred_element_type=f32).astype(bf16)

        gemm_chunk(my)

        barrier = pltpu.get_barrier_semaphore()
        for nbr in (left, right):
            pl.semaphore_signal(barrier, inc=1, device_id=(nbr,),
                                device_id_type=pl.DeviceIdType.MESH)
        pl.semaphore_wait(barrier, 2)

        def store_out(idx, co, val_f32):
            g = gate_ref[0, co:co + Dh_].astype(f32)
            r = res_ref[pl.ds(idx * C, C), co:co + Dh_].astype(f32)
            out_ref[pl.ds(idx * C, C), co:co + Dh_] = (
                r + g[None, :] * val_f32
            ).astype(bf16)

        def hop(src, dst, ssem, rsem, tgt):
            return pltpu.make_async_remote_copy(
                src_ref=src, dst_ref=dst, send_sem=ssem, recv_sem=rsem,
                device_id=(tgt,), device_id_type=pl.DeviceIdType.MESH,
            )

        rs_send_r[0, :, :] = p_ref[pl.ds(my * C, C), 0:Dh_]
        rs_send_l[0, :, :] = p_ref[pl.ds(my * C, C), Dh_:D]
        for s in range(N_DEV - 1):
            rr = hop(rs_send_r.at[s], rs_recv_r.at[s], ssem_r.at[s],
                     rsem_r.at[s], right)
            rl = hop(rs_send_l.at[s], rs_recv_l.at[s], ssem_l.at[s],
                     rsem_l.at[s], left)
            rr.start()
            rl.start()
            if s == 0:
                gemm_chunk(lax.rem(my + N_DEV - 1, N_DEV))
                gemm_chunk(lax.rem(my + 1, N_DEV))
            elif s == 1:
                gemm_chunk(lax.rem(my + 2, N_DEV))
            rr.wait()
            rl.wait()
            idx_r = lax.rem(my - s - 1 + 2 * N_DEV, N_DEV)
            idx_l = lax.rem(my + s + 1, N_DEV)
            loc_r = p_ref[pl.ds(idx_r * C, C), 0:Dh_]
            loc_l = p_ref[pl.ds(idx_l * C, C), Dh_:D]
            if s < N_DEV - 2:
                rs_send_r[s + 1, :, :] = rs_recv_r[s, :, :] + loc_r
                rs_send_l[s + 1, :, :] = rs_recv_l[s, :, :] + loc_l
            else:
                red_r = rs_recv_r[s, :, :].astype(f32) + loc_r.astype(f32)
                red_l = rs_recv_l[s, :, :].astype(f32) + loc_l.astype(f32)
                store_out(idx_r, 0, red_r)
                store_out(idx_l, Dh_, red_l)
                ag_r[0, :, :] = red_r.astype(bf16)
                ag_l[0, :, :] = red_l.astype(bf16)

        for t in range(N_DEV - 1):
            rr = hop(ag_r.at[t], ag_r.at[t + 1], ssem_r.at[N_DEV - 1 + t],
                     rsem_r.at[N_DEV - 1 + t], right)
            rl = hop(ag_l.at[t], ag_l.at[t + 1], ssem_l.at[N_DEV - 1 + t],
                     rsem_l.at[N_DEV - 1 + t], left)
            rr.start()
            rl.start()
            rr.wait()
            rl.wait()
            idx_r = lax.rem(my - t + 2 * N_DEV, N_DEV)
            idx_l = lax.rem(my + t, N_DEV)
            store_out(idx_r, 0, ag_r[t + 1, :, :].astype(f32))
            store_out(idx_l, Dh_, ag_l[t + 1, :, :].astype(f32))

    n_hops = 2 * (N_DEV - 1)
    n_in = 3 if Wo is None else 4
    extra_scratch = [] if Wo is None else [pltpu.VMEM((S, D), bf16)]
    args = (p, res, gate) if Wo is None else (p, Wo, res, gate)
    return pl.pallas_call(
        body,
        out_shape=jax.ShapeDtypeStruct((S, D), bf16),
        in_specs=[pl.BlockSpec(memory_space=pltpu.VMEM)] * n_in,
        out_specs=pl.BlockSpec(memory_space=pltpu.VMEM),
        scratch_shapes=extra_scratch + [
            pltpu.VMEM((N_DEV - 1, C, Dh_), bf16),
            pltpu.VMEM((N_DEV - 1, C, Dh_), bf16),
            pltpu.VMEM((N_DEV, C, Dh_), bf16),
            pltpu.VMEM((N_DEV - 1, C, Dh_), bf16),
            pltpu.VMEM((N_DEV - 1, C, Dh_), bf16),
            pltpu.VMEM((N_DEV, C, Dh_), bf16),
            pltpu.SemaphoreType.DMA((n_hops,)),
            pltpu.SemaphoreType.DMA((n_hops,)),
            pltpu.SemaphoreType.DMA((n_hops,)),
            pltpu.SemaphoreType.DMA((n_hops,)),
        ],
        compiler_params=pltpu.CompilerParams(
            collective_id=collective_id,
            vmem_limit_bytes=61 * 2**20,
        ),
    )(*args)


def _ln_qkv(x0, sa, sha, Wq, Wk, Wv):
    S, D = x0.shape
    BR = 1024
    eps = 1e-5

    def body(x_ref, sa_ref, sha_ref, wq_ref, wk_ref, wv_ref,
             q_ref, k_ref, v_ref):
        xb = x_ref[...]
        m = jnp.mean(xb, axis=-1, keepdims=True)
        c = xb - m
        v_ = jnp.mean(c * c, axis=-1, keepdims=True)
        g = (1.0 + sa_ref[0, :])[None, :]
        xm = (c * lax.rsqrt(v_ + eps) * g + sha_ref[0, :][None, :]
              ).astype(bf16)
        for w_ref, o_ref in ((wq_ref, q_ref), (wk_ref, k_ref),
                             (wv_ref, v_ref)):
            o_ref[...] = lax.dot_general(
                xm, w_ref[...], (((1,), (0,)), ((), ())),
                preferred_element_type=f32).astype(bf16)

    out = pl.pallas_call(
        body,
        grid=(S // BR,),
        in_specs=[
            pl.BlockSpec((BR, D), lambda i: (i, 0)),
            pl.BlockSpec((1, D), lambda i: (0, 0)),
            pl.BlockSpec((1, D), lambda i: (0, 0)),
            pl.BlockSpec((D, D), lambda i: (0, 0)),
            pl.BlockSpec((D, D), lambda i: (0, 0)),
            pl.BlockSpec((D, D), lambda i: (0, 0)),
        ],
        out_specs=[pl.BlockSpec((BR, D), lambda i: (i, 0))] * 3,
        out_shape=[jax.ShapeDtypeStruct((S, D), bf16)] * 3,
    )(x0, sa, sha, Wq, Wk, Wv)
    return out


def _ln_ffn(x1, sm, shm, Wf1, Wf2):
    S, D = x1.shape
    F = Wf1.shape[1]
    BR = 1024
    eps = 1e-5

    def body(x_ref, sm_ref, shm_ref, w1_ref, w2_ref, o_ref):
        xb = x_ref[...].astype(f32)
        m = jnp.mean(xb, axis=-1, keepdims=True)
        c = xb - m
        v_ = jnp.mean(c * c, axis=-1, keepdims=True)
        g = (1.0 + sm_ref[0, :])[None, :]
        xm = (c * lax.rsqrt(v_ + eps) * g + shm_ref[0, :][None, :]
              ).astype(bf16)
        h = lax.dot_general(xm, w1_ref[...], (((1,), (0,)), ((), ())),
                            preferred_element_type=f32)
        h = (h * jax.nn.sigmoid(h)).astype(bf16)
        o_ref[...] = lax.dot_general(
            h, w2_ref[...], (((1,), (0,)), ((), ())),
            preferred_element_type=f32).astype(bf16)

    return pl.pallas_call(
        body,
        grid=(S // BR,),
        in_specs=[
            pl.BlockSpec((BR, D), lambda i: (i, 0)),
            pl.BlockSpec((1, D), lambda i: (0, 0)),
            pl.BlockSpec((1, D), lambda i: (0, 0)),
            pl.BlockSpec((D, F), lambda i: (0, 0)),
            pl.BlockSpec((F, D), lambda i: (0, 0)),
        ],
        out_specs=pl.BlockSpec((BR, D), lambda i: (i, 0)),
        out_shape=jax.ShapeDtypeStruct((S, D), bf16),
    )(x1, sm, shm, Wf1, Wf2)


def _attention(q, k, v):
    S, HD = q.shape
    Dh = 128
    H = HD // Dh
    scale = 1.0 / (Dh ** 0.5)
    BQ = 1024

    def body(q_ref, k_ref, v_ref, o_ref):
        qb = q_ref[...]
        kb = k_ref[...]
        vb = v_ref[...]
        s = lax.dot_general(qb, kb, (((1,), (1,)), ((), ())),
                            preferred_element_type=f32) * scale
        p = jnp.exp(s)
        l = jnp.sum(p, axis=-1, keepdims=True)
        o = lax.dot_general(p.astype(bf16), vb, (((1,), (0,)), ((), ())),
                            preferred_element_type=f32)
        o_ref[...] = (o / l).astype(bf16)

    return pl.pallas_call(
        body,
        grid=(H, S // BQ),
        in_specs=[
            pl.BlockSpec((BQ, Dh), lambda h, i: (i, h)),
            pl.BlockSpec((S, Dh), lambda h, i: (0, h)),
            pl.BlockSpec((S, Dh), lambda h, i: (0, h)),
        ],
        out_specs=pl.BlockSpec((BQ, Dh), lambda h, i: (i, h)),
        out_shape=jax.ShapeDtypeStruct((S, HD), bf16),
    )(q, k, v)


def kernel(x, Wq, Wk, Wv, Wo, t_emb, W_mod, W_ff1, W_ff2):
    x0 = x[0].astype(f32)

    mod = (t_emb.astype(f32) @ W_mod.astype(f32))
    sa, sha, ga, sm, shm, gm = jnp.split(mod, 6, axis=-1)

    q, k, v = _ln_qkv(x0, sa, sha, Wq.astype(bf16), Wk.astype(bf16),
                      Wv.astype(bf16))
    attn = _attention(q, k, v)

    x1 = _ring_allreduce_fused(attn, x0.astype(bf16), ga, collective_id=0,
                               Wo=Wo.astype(bf16))

    partial2 = _ln_ffn(x1, sm, shm, W_ff1.astype(bf16), W_ff2.astype(bf16))
    out = _ring_allreduce_fused(partial2, x1, gm, collective_id=1)

    return out[None].astype(f32)


# device time: 414152 ns/iter; 1.0770x vs baseline; 1.0770x over previous
import functools

import jax
import jax.numpy as jnp
from jax import lax
from jax.experimental import pallas as pl
from jax.experimental.pallas import tpu as pltpu

N_DEV = 4
USE_FLASH = True

f32 = jnp.float32
bf16 = jnp.bfloat16


def _ring_allreduce_fused(p, res, gate, collective_id, Wo=None):
    S, D = p.shape
    C = S // N_DEV
    Dh_ = D // 2

    def body(*refs):
        if Wo is None:
            (p_ref, res_ref, gate_ref, out_ref,
             rs_send_r, rs_recv_r, ag_r,
             rs_send_l, rs_recv_l, ag_l,
             ssem_r, rsem_r, ssem_l, rsem_l) = refs
        else:
            (a_ref, wo_ref, res_ref, gate_ref, out_ref, p_scr,
             rs_send_r, rs_recv_r, ag_r,
             rs_send_l, rs_recv_l, ag_l,
             ssem_r, rsem_r, ssem_l, rsem_l) = refs
            p_ref = p_scr
        my = lax.axis_index("i")
        left = lax.rem(my + N_DEV - 1, N_DEV)
        right = lax.rem(my + 1, N_DEV)

        def gemm_chunk(idx):
            if Wo is None:
                return
            p_scr[pl.ds(idx * C, C), :] = lax.dot_general(
                a_ref[pl.ds(idx * C, C), :], wo_ref[...],
                (((1,), (0,)), ((), ())),
                preferred_element_type=f32).astype(bf16)

        gemm_chunk(my)

        barrier = pltpu.get_barrier_semaphore()
        for nbr in (left, right):
            pl.semaphore_signal(barrier, inc=1, device_id=(nbr,),
                                device_id_type=pl.DeviceIdType.MESH)
        pl.semaphore_wait(barrier, 2)

        def store_out(idx, co, val_f32):
            g = gate_ref[0, co:co + Dh_].astype(f32)
            r = res_ref[pl.ds(idx * C, C), co:co + Dh_].astype(f32)
            out_ref[pl.ds(idx * C, C), co:co + Dh_] = (
                r + g[None, :] * val_f32
            ).astype(bf16)

        def hop(src, dst, ssem, rsem, tgt):
            return pltpu.make_async_remote_copy(
                src_ref=src, dst_ref=dst, send_sem=ssem, recv_sem=rsem,
                device_id=(tgt,), device_id_type=pl.DeviceIdType.MESH,
            )

        rs_send_r[0, :, :] = p_ref[pl.ds(my * C, C), 0:Dh_]
        rs_send_l[0, :, :] = p_ref[pl.ds(my * C, C), Dh_:D]
        for s in range(N_DEV - 1):
            rr = hop(rs_send_r.at[s], rs_recv_r.at[s], ssem_r.at[s],
                     rsem_r.at[s], right)
            rl = hop(rs_send_l.at[s], rs_recv_l.at[s], ssem_l.at[s],
                     rsem_l.at[s], left)
            rr.start()
            rl.start()
            if s == 0:
                gemm_chunk(lax.rem(my + N_DEV - 1, N_DEV))
                gemm_chunk(lax.rem(my + 1, N_DEV))
            elif s == 1:
                gemm_chunk(lax.rem(my + 2, N_DEV))
            rr.wait()
            rl.wait()
            idx_r = lax.rem(my - s - 1 + 2 * N_DEV, N_DEV)
            idx_l = lax.rem(my + s + 1, N_DEV)
            loc_r = p_ref[pl.ds(idx_r * C, C), 0:Dh_]
            loc_l = p_ref[pl.ds(idx_l * C, C), Dh_:D]
            if s < N_DEV - 2:
                rs_send_r[s + 1, :, :] = rs_recv_r[s, :, :] + loc_r
                rs_send_l[s + 1, :, :] = rs_recv_l[s, :, :] + loc_l
            else:
                red_r = rs_recv_r[s, :, :].astype(f32) + loc_r.astype(f32)
                red_l = rs_recv_l[s, :, :].astype(f32) + loc_l.astype(f32)
                store_out(idx_r, 0, red_r)
                store_out(idx_l, Dh_, red_l)
                ag_r[0, :, :] = red_r.astype(bf16)
                ag_l[0, :, :] = red_l.astype(bf16)

        for t in range(N_DEV - 1):
            rr = hop(ag_r.at[t], ag_r.at[t + 1], ssem_r.at[N_DEV - 1 + t],
                     rsem_r.at[N_DEV - 1 + t], right)
            rl = hop(ag_l.at[t], ag_l.at[t + 1], ssem_l.at[N_DEV - 1 + t],
                     rsem_l.at[N_DEV - 1 + t], left)
            rr.start()
            rl.start()
            rr.wait()
            rl.wait()
            idx_r = lax.rem(my - t + 2 * N_DEV, N_DEV)
            idx_l = lax.rem(my + t, N_DEV)
            store_out(idx_r, 0, ag_r[t + 1, :, :].astype(f32))
            store_out(idx_l, Dh_, ag_l[t + 1, :, :].astype(f32))

    n_hops = 2 * (N_DEV - 1)
    n_in = 3 if Wo is None else 4
    extra_scratch = [] if Wo is None else [pltpu.VMEM((S, D), bf16)]
    args = (p, res, gate) if Wo is None else (p, Wo, res, gate)
    return pl.pallas_call(
        body,
        out_shape=jax.ShapeDtypeStruct((S, D), bf16),
        in_specs=[pl.BlockSpec(memory_space=pltpu.VMEM)] * n_in,
        out_specs=pl.BlockSpec(memory_space=pltpu.VMEM),
        scratch_shapes=extra_scratch + [
            pltpu.VMEM((N_DEV - 1, C, Dh_), bf16),
            pltpu.VMEM((N_DEV - 1, C, Dh_), bf16),
            pltpu.VMEM((N_DEV, C, Dh_), bf16),
            pltpu.VMEM((N_DEV - 1, C, Dh_), bf16),
            pltpu.VMEM((N_DEV - 1, C, Dh_), bf16),
            pltpu.VMEM((N_DEV, C, Dh_), bf16),
            pltpu.SemaphoreType.DMA((n_hops,)),
            pltpu.SemaphoreType.DMA((n_hops,)),
            pltpu.SemaphoreType.DMA((n_hops,)),
            pltpu.SemaphoreType.DMA((n_hops,)),
        ],
        compiler_params=pltpu.CompilerParams(
            collective_id=collective_id,
            **({} if Wo is None else {"vmem_limit_bytes": 61 * 2**20}),
        ),
    )(*args)


def _hd_allreduce_fused(p, res, gate, collective_id):
    S, D = p.shape
    C = S // N_DEV
    HALF = S // 2
    W = D // 2

    def body(p_ref, res_ref, gate_ref, out_ref,
             hsA, hrecvA, qrecvA, hsB, hrecvB, qrecvB,
             ssemA, rsemA, ssemB, rsemB):
        my = lax.axis_index("i")
        p1 = jnp.bitwise_xor(my, 1)
        p2 = 3 - my

        def sel4(a, b, c, d):
            return jnp.where(my == 0, a, jnp.where(my == 1, b,
                             jnp.where(my == 2, c, d)))

        mA_my = sel4(0, 2, 3, 1)
        mA_pb = sel4(1, 3, 2, 0)
        mB_my = sel4(0, 1, 3, 2)
        mB_pb = sel4(1, 0, 2, 3)
        keepA = jnp.where(mA_my < 2, 0, HALF)
        keepB = jnp.where(mB_my < 2, 0, HALF)

        barrier = pltpu.get_barrier_semaphore()
        for nbr in (p1, p2):
            pl.semaphore_signal(barrier, inc=1, device_id=(nbr,),
                                device_id_type=pl.DeviceIdType.MESH)
        pl.semaphore_wait(barrier, 2)

        def mk(src, dst, ssem, rsem, tgt):
            return pltpu.make_async_remote_copy(
                src_ref=src, dst_ref=dst, send_sem=ssem, recv_sem=rsem,
                device_id=(tgt,), device_id_type=pl.DeviceIdType.MESH,
            )

        r1A = mk(p_ref.at[pl.ds(HALF - keepA, HALF), 0:W], hrecvA,
                 ssemA.at[0], rsemA.at[0], p1)
        r1B = mk(p_ref.at[pl.ds(HALF - keepB, HALF), W:D], hrecvB,
                 ssemB.at[0], rsemB.at[0], p2)
        r1A.start()
        r1B.start()
        r1A.wait()
        r1B.wait()
        hsA[...] = p_ref[pl.ds(keepA, HALF), 0:W] + hrecvA[...]
        hsB[...] = p_ref[pl.ds(keepB, HALF), W:D] + hrecvB[...]

        r2A = mk(hsA.at[pl.ds(mA_pb * C - keepA, C), :], qrecvA,
                 ssemA.at[1], rsemA.at[1], p2)
        r2B = mk(hsB.at[pl.ds(mB_pb * C - keepB, C), :], qrecvB,
                 ssemB.at[1], rsemB.at[1], p1)
        r2A.start()
        r2B.start()
        r2A.wait()
        r2B.wait()

        def store(m_my, keep, co, hs, qrecv):
            red = (hs[pl.ds(m_my * C - keep, C), :].astype(f32)
                   + qrecv[...].astype(f32))
            g = gate_ref[0, co:co + W].astype(f32)
            r_ = res_ref[pl.ds(m_my * C, C), co:co + W].astype(f32)
            out_ref[pl.ds(m_my * C, C), co:co + W] = (
                r_ + g[None, :] * red).astype(bf16)

        store(mA_my, keepA, 0, hsA, qrecvA)
        store(mB_my, keepB, W, hsB, qrecvB)

        r3A = mk(out_ref.at[pl.ds(mA_my * C, C), 0:W],
                 out_ref.at[pl.ds(mA_my * C, C), 0:W],
                 ssemA.at[2], rsemA.at[2], p2)
        r3B = mk(out_ref.at[pl.ds(mB_my * C, C), W:D],
                 out_ref.at[pl.ds(mB_my * C, C), W:D],
                 ssemB.at[2], rsemB.at[2], p1)
        r3A.start()
        r3B.start()
        r3A.wait()
        r3B.wait()

        r4A = mk(out_ref.at[pl.ds(keepA, HALF), 0:W],
                 out_ref.at[pl.ds(keepA, HALF), 0:W],
                 ssemA.at[3], rsemA.at[3], p1)
        r4B = mk(out_ref.at[pl.ds(keepB, HALF), W:D],
                 out_ref.at[pl.ds(keepB, HALF), W:D],
                 ssemB.at[3], rsemB.at[3], p2)
        r4A.start()
        r4B.start()
        r4A.wait()
        r4B.wait()

    return pl.pallas_call(
        body,
        out_shape=jax.ShapeDtypeStruct((S, D), bf16),
        in_specs=[pl.BlockSpec(memory_space=pltpu.VMEM)] * 3,
        out_specs=pl.BlockSpec(memory_space=pltpu.VMEM),
        scratch_shapes=[
            pltpu.VMEM((HALF, W), bf16),
            pltpu.VMEM((HALF, W), bf16),
            pltpu.VMEM((C, W), bf16),
            pltpu.VMEM((HALF, W), bf16),
            pltpu.VMEM((HALF, W), bf16),
            pltpu.VMEM((C, W), bf16),
            pltpu.SemaphoreType.DMA((4,)),
            pltpu.SemaphoreType.DMA((4,)),
            pltpu.SemaphoreType.DMA((4,)),
            pltpu.SemaphoreType.DMA((4,)),
        ],
        compiler_params=pltpu.CompilerParams(collective_id=collective_id),
    )(p, res, gate)


def _ln_qkv(x0, sa, sha, Wq, Wk, Wv):
    S, D = x0.shape
    BR = 1024
    eps = 1e-5

    def body(x_ref, sa_ref, sha_ref, wq_ref, wk_ref, wv_ref,
             q_ref, k_ref, v_ref):
        xb = x_ref[...]
        m = jnp.mean(xb, axis=-1, keepdims=True)
        c = xb - m
        v_ = jnp.mean(c * c, axis=-1, keepdims=True)
        g = (1.0 + sa_ref[0, :])[None, :]
        xm = (c * lax.rsqrt(v_ + eps) * g + sha_ref[0, :][None, :]
              ).astype(bf16)
        for w_ref, o_ref in ((wq_ref, q_ref), (wk_ref, k_ref),
                             (wv_ref, v_ref)):
            o_ref[...] = lax.dot_general(
                xm, w_ref[...], (((1,), (0,)), ((), ())),
                preferred_element_type=f32).astype(bf16)

    out = pl.pallas_call(
        body,
        grid=(S // BR,),
        in_specs=[
            pl.BlockSpec((BR, D), lambda i: (i, 0)),
            pl.BlockSpec((1, D), lambda i: (0, 0)),
            pl.BlockSpec((1, D), lambda i: (0, 0)),
            pl.BlockSpec((D, D), lambda i: (0, 0)),
            pl.BlockSpec((D, D), lambda i: (0, 0)),
            pl.BlockSpec((D, D), lambda i: (0, 0)),
        ],
        out_specs=[pl.BlockSpec((BR, D), lambda i: (i, 0))] * 3,
        out_shape=[jax.ShapeDtypeStruct((S, D), bf16)] * 3,
    )(x0, sa, sha, Wq, Wk, Wv)
    return out


def _ln_ffn(x1, sm, shm, Wf1, Wf2):
    S, D = x1.shape
    F = Wf1.shape[1]
    BR = 1024
    eps = 1e-5

    def body(x_ref, sm_ref, shm_ref, w1_ref, w2_ref, o_ref):
        xb = x_ref[...].astype(f32)
        m = jnp.mean(xb, axis=-1, keepdims=True)
        c = xb - m
        v_ = jnp.mean(c * c, axis=-1, keepdims=True)
        g = (1.0 + sm_ref[0, :])[None, :]
        xm = (c * lax.rsqrt(v_ + eps) * g + shm_ref[0, :][None, :]
              ).astype(bf16)
        h = lax.dot_general(xm, w1_ref[...], (((1,), (0,)), ((), ())),
                            preferred_element_type=f32)
        h = (h * jax.nn.sigmoid(h)).astype(bf16)
        o_ref[...] = lax.dot_general(
            h, w2_ref[...], (((1,), (0,)), ((), ())),
            preferred_element_type=f32).astype(bf16)

    return pl.pallas_call(
        body,
        grid=(S // BR,),
        in_specs=[
            pl.BlockSpec((BR, D), lambda i: (i, 0)),
            pl.BlockSpec((1, D), lambda i: (0, 0)),
            pl.BlockSpec((1, D), lambda i: (0, 0)),
            pl.BlockSpec((D, F), lambda i: (0, 0)),
            pl.BlockSpec((F, D), lambda i: (0, 0)),
        ],
        out_specs=pl.BlockSpec((BR, D), lambda i: (i, 0)),
        out_shape=jax.ShapeDtypeStruct((S, D), bf16),
    )(x1, sm, shm, Wf1, Wf2)


def _attention(q, k, v):
    S, HD = q.shape
    Dh = 128
    H = HD // Dh
    scale = 1.0 / (Dh ** 0.5)
    BQ = 1024

    def body(q_ref, k_ref, v_ref, o_ref):
        qb = q_ref[...]
        kb = k_ref[...]
        vb = v_ref[...]
        s = lax.dot_general(qb, kb, (((1,), (1,)), ((), ())),
                            preferred_element_type=f32) * scale
        p = jnp.exp(s)
        l = jnp.sum(p, axis=-1, keepdims=True)
        o = lax.dot_general(p.astype(bf16), vb, (((1,), (0,)), ((), ())),
                            preferred_element_type=f32)
        o_ref[...] = (o / l).astype(bf16)

    return pl.pallas_call(
        body,
        grid=(H, S // BQ),
        in_specs=[
            pl.BlockSpec((BQ, Dh), lambda h, i: (i, h)),
            pl.BlockSpec((S, Dh), lambda h, i: (0, h)),
            pl.BlockSpec((S, Dh), lambda h, i: (0, h)),
        ],
        out_specs=pl.BlockSpec((BQ, Dh), lambda h, i: (i, h)),
        out_shape=jax.ShapeDtypeStruct((S, HD), bf16),
    )(q, k, v)


def kernel(x, Wq, Wk, Wv, Wo, t_emb, W_mod, W_ff1, W_ff2):
    x0 = x[0].astype(f32)

    mod = (t_emb.astype(f32) @ W_mod.astype(f32))
    sa, sha, ga, sm, shm, gm = jnp.split(mod, 6, axis=-1)

    q, k, v = _ln_qkv(x0, sa, sha, Wq.astype(bf16), Wk.astype(bf16),
                      Wv.astype(bf16))
    attn = _attention(q, k, v)

    partial = (attn @ Wo.astype(bf16)).astype(bf16)
    x1 = _hd_allreduce_fused(partial, x0.astype(bf16), ga, collective_id=0)

    partial2 = _ln_ffn(x1, sm, shm, W_ff1.astype(bf16), W_ff2.astype(bf16))
    out = _hd_allreduce_fused(partial2, x1, gm, collective_id=1)

    return out[None].astype(f32)
